# baseline (speedup 1.0000x reference)
import math

import jax
import jax.numpy as jnp
from jax.experimental import pallas as pl
from jax.experimental.pallas import tpu as pltpu

_S = 512
_K = 16
_NEG = float(-3e38)

_INTERPRET = False


def _mm_t(a, b):
    return jax.lax.dot_general(a, b, (((1,), (1,)), ((), ())),
                               preferred_element_type=jnp.float32)


def _fps_body(x0_ref, x1_ref, x2_ref, o0_ref, o1_ref, o2_ref, dists_ref):
    Bc = x0_ref.shape[1]
    X0 = x0_ref[...]
    X1 = x1_ref[...]
    X2 = x2_ref[...]
    ri = (jax.lax.broadcasted_iota(jnp.int32, (128, 1, 128), 0) * 128
          + jax.lax.broadcasted_iota(jnp.int32, (128, 1, 128), 2))
    dists_ref[...] = jnp.full((128, Bc, 128), 1e10, dtype=jnp.float32)

    def body(i, far):
        msk = (ri == far.reshape(1, Bc, 1)).astype(jnp.float32)
        c0 = jnp.sum(jnp.sum(X0 * msk, axis=2), axis=0)
        c1 = jnp.sum(jnp.sum(X1 * msk, axis=2), axis=0)
        c2 = jnp.sum(jnp.sum(X2 * msk, axis=2), axis=0)
        o0_ref[0, pl.ds(i, 1), :] = c0.reshape(1, Bc)
        o1_ref[0, pl.ds(i, 1), :] = c1.reshape(1, Bc)
        o2_ref[0, pl.ds(i, 1), :] = c2.reshape(1, Bc)
        d0 = X0 - c0.reshape(1, Bc, 1)
        d1 = X1 - c1.reshape(1, Bc, 1)
        d2 = X2 - c2.reshape(1, Bc, 1)
        d = d0 * d0 + d1 * d1 + d2 * d2
        nd = jnp.minimum(dists_ref[...], d)
        dists_ref[...] = nd
        mx = jnp.max(jnp.max(nd, axis=2), axis=0)
        cand = jnp.where(nd == mx.reshape(1, Bc, 1), ri, jnp.int32(1 << 30))
        far2 = jnp.min(jnp.min(cand, axis=2), axis=0)
        return far2

    jax.lax.fori_loop(0, _S, body, jnp.zeros((Bc,), jnp.int32))


def _top16_idx(pdp):
    S = pdp.shape[0]
    lane = jax.lax.broadcasted_iota(jnp.int32, (S, S), 1)
    kcol = jax.lax.broadcasted_iota(jnp.int32, (S, _K), 1)

    def round_fn(r, state):
        work, idxs = state
        m = jnp.max(work, axis=1, keepdims=True)
        cand = jnp.where(work == m, lane, jnp.int32(1 << 30))
        j = jnp.min(cand, axis=1, keepdims=True)
        idxs = jnp.where(kcol == r, j, idxs)
        work = jnp.where(lane == j, _NEG, work)
        return work, idxs

    _, idxs = jax.lax.fori_loop(
        0, _K, round_fn, (pdp, jnp.zeros((S, _K), jnp.int32)))
    return idxs


def _layer(xt, P, Wt, Wd, d):
    S = xt.shape[0]
    ones = jnp.ones((S, 1), jnp.float32)
    xx = jnp.sum(xt * xt, axis=1, keepdims=True)
    A = jnp.concatenate([2.0 * xt, ones], axis=1)
    Bm = jnp.concatenate([xt, -xx], axis=1)
    pdp = _mm_t(A, Bm)
    idxs = _top16_idx(pdp)

    Y = jnp.dot(xt, Wt, preferred_element_type=jnp.float32)
    Z = jnp.dot(xt, Wd, preferred_element_type=jnp.float32)
    Yq, Yk, Yv = Y[:, :d], Y[:, d:2 * d], Y[:, 2 * d:]
    Zq, Zk, Zv = Z[:, :d], Z[:, d:2 * d], Z[:, 2 * d:]
    tab = jnp.concatenate([Yq * Yk, Yq, Yk, Yv], axis=1)
    lane = jax.lax.broadcasted_iota(jnp.int32, (S, S), 1)
    w1 = Zk + P
    inv = jnp.float32(1.0 / math.sqrt(d))
    logits_k = []
    yvg_k = []
    for r in range(_K):
        oh = (lane == idxs[:, r:r + 1]).astype(jnp.float32)
        g = jnp.dot(oh, tab, preferred_element_type=jnp.float32)
        logits_k.append((g[:, :d] + g[:, d:2 * d] * w1
                         + g[:, 2 * d:3 * d] * Zq) * inv)
        yvg_k.append(g[:, 3 * d:])
    mrow = logits_k[0]
    for r in range(1, _K):
        mrow = jnp.maximum(mrow, logits_k[r])
    es = [jnp.exp(lg - mrow) for lg in logits_k]
    ssum = es[0]
    for r in range(1, _K):
        ssum = ssum + es[r]
    acc = es[0] * yvg_k[0]
    for r in range(1, _K):
        acc = acc + es[r] * yvg_k[r]
    return acc / ssum + Zv + P


def _layers_body(pt_ref, wt1, wd1, wp1, wt2, wd2, wp2, wt3, wd3, wp3, out_ref):
    pt = pt_ref[0]
    P1 = jnp.dot(pt, wp1[...], preferred_element_type=jnp.float32)
    x1 = _layer(pt, P1, wt1[...], wd1[...], 64)
    P2 = jnp.dot(pt, wp2[...], preferred_element_type=jnp.float32)
    x2 = _layer(x1, P2, wt2[...], wd2[...], 64)
    P3 = jnp.dot(pt, wp3[...], preferred_element_type=jnp.float32)
    x3 = _layer(x2, P3, wt3[...], wd3[...], 128)
    feat = jnp.concatenate([x1, x2, x3], axis=1)
    mx = jnp.max(feat, axis=0)
    av = jnp.sum(feat, axis=0) * jnp.float32(1.0 / _S)
    out_ref[0, 0] = jnp.concatenate([mx, av])


def _prep_w(Wq, Wk, Wv, C, pad_to):
    top = jnp.concatenate([Wq[:C], Wk[:C], Wv[:C]], axis=1)
    dif = jnp.concatenate([Wq[C:] - Wq[:C], Wk[C:] - Wk[:C], Wv[C:] - Wv[:C]],
                          axis=1)
    if pad_to > C:
        z = jnp.zeros((pad_to - C, top.shape[1]), jnp.float32)
        top = jnp.concatenate([top, z], axis=0)
        dif = jnp.concatenate([dif, z], axis=0)
    return top, dif


def _pad_rows(W, pad_to):
    return jnp.concatenate(
        [W, jnp.zeros((pad_to - W.shape[0], W.shape[1]), jnp.float32)], axis=0)


def kernel(x, Wq1, Wk1, Wv1, Wp1, Wq2, Wk2, Wv2, Wp2, Wq3, Wk3, Wv3, Wp3):
    B, N, _ = x.shape
    planes = [x[:, :, c].reshape(B, 128, 128).transpose(1, 0, 2)
              for c in range(3)]
    nc = 2
    Bc = B // nc
    o0, o1, o2 = pl.pallas_call(
        _fps_body,
        grid=(nc,),
        in_specs=[pl.BlockSpec((128, Bc, 128), lambda j: (0, j, 0))] * 3,
        out_specs=[pl.BlockSpec((1, _S, Bc), lambda j: (j, 0, 0))] * 3,
        out_shape=[jax.ShapeDtypeStruct((nc, _S, Bc), jnp.float32)] * 3,
        scratch_shapes=[pltpu.VMEM((128, Bc, 128), jnp.float32)],
        compiler_params=pltpu.CompilerParams(
            dimension_semantics=("parallel",)),
        interpret=_INTERPRET,
    )(*planes)
    o0, o1, o2 = (o.transpose(1, 0, 2).reshape(_S, B) for o in (o0, o1, o2))
    pt = jnp.stack([o0, o1, o2], axis=-1).transpose(1, 0, 2)
    ptp = jnp.concatenate([pt, jnp.zeros((B, _S, 5), jnp.float32)], axis=-1)

    wt1, wd1 = _prep_w(Wq1, Wk1, Wv1, 3, 8)
    wt2, wd2 = _prep_w(Wq2, Wk2, Wv2, 64, 64)
    wt3, wd3 = _prep_w(Wq3, Wk3, Wv3, 64, 64)
    wp1 = _pad_rows(Wp1, 8)
    wp2 = _pad_rows(Wp2, 8)
    wp3 = _pad_rows(Wp3, 8)

    def wspec(w):
        return pl.BlockSpec(w.shape, lambda b: (0, 0))

    out = pl.pallas_call(
        _layers_body,
        grid=(B,),
        in_specs=[pl.BlockSpec((1, _S, 8), lambda b: (b, 0, 0)),
                  wspec(wt1), wspec(wd1), wspec(wp1),
                  wspec(wt2), wspec(wd2), wspec(wp2),
                  wspec(wt3), wspec(wd3), wspec(wp3)],
        out_specs=pl.BlockSpec((1, 1, _S), lambda b: (b, 0, 0)),
        out_shape=jax.ShapeDtypeStruct((B, 1, _S), jnp.float32),
        compiler_params=pltpu.CompilerParams(
            dimension_semantics=("parallel",)),
        interpret=_INTERPRET,
    )(ptp, wt1, wd1, wp1, wt2, wd2, wp2, wt3, wd3, wp3)
    return out.reshape(B, _S)


# speedup vs baseline: 1.1405x; 1.1405x over previous
import math

import jax
import jax.numpy as jnp
from jax.experimental import pallas as pl
from jax.experimental.pallas import tpu as pltpu

_S = 512
_K = 16
_NEG = float(-3e38)

_INTERPRET = False


def _mm_t(a, b):
    return jax.lax.dot_general(a, b, (((1,), (1,)), ((), ())),
                               preferred_element_type=jnp.float32)


def _fps_body(x0_ref, x1_ref, x2_ref, o0_ref, o1_ref, o2_ref, dists_ref):
    Bc = x0_ref.shape[1]
    X0 = x0_ref[...]
    X1 = x1_ref[...]
    X2 = x2_ref[...]
    ri = (jax.lax.broadcasted_iota(jnp.int32, (128, 1, 128), 0) * 128
          + jax.lax.broadcasted_iota(jnp.int32, (128, 1, 128), 2))
    dists_ref[...] = jnp.full((128, Bc, 128), 1e10, dtype=jnp.float32)

    UNROLL = 4

    def body(i, far):
        for r in range(UNROLL):
            msk = (ri == far.reshape(1, Bc, 1)).astype(jnp.float32)
            c0 = jnp.sum(jnp.sum(X0 * msk, axis=2), axis=0)
            c1 = jnp.sum(jnp.sum(X1 * msk, axis=2), axis=0)
            c2 = jnp.sum(jnp.sum(X2 * msk, axis=2), axis=0)
            step = i * UNROLL + r
            o0_ref[0, pl.ds(step, 1), :] = c0.reshape(1, Bc)
            o1_ref[0, pl.ds(step, 1), :] = c1.reshape(1, Bc)
            o2_ref[0, pl.ds(step, 1), :] = c2.reshape(1, Bc)
            d0 = X0 - c0.reshape(1, Bc, 1)
            d1 = X1 - c1.reshape(1, Bc, 1)
            d2 = X2 - c2.reshape(1, Bc, 1)
            d = d0 * d0 + d1 * d1 + d2 * d2
            nd = jnp.minimum(dists_ref[...], d)
            dists_ref[...] = nd
            mx = jnp.max(jnp.max(nd, axis=2), axis=0)
            cand = jnp.where(nd == mx.reshape(1, Bc, 1), ri,
                             jnp.int32(1 << 30))
            far = jnp.min(jnp.min(cand, axis=2), axis=0)
        return far

    jax.lax.fori_loop(0, _S // UNROLL, body, jnp.zeros((Bc,), jnp.int32))


def _top16_idx(pdp):
    S = pdp.shape[0]
    lane = jax.lax.broadcasted_iota(jnp.int32, (S, S), 1)
    kcol = jax.lax.broadcasted_iota(jnp.int32, (S, _K), 1)

    work = pdp
    idxs = jnp.zeros((S, _K), jnp.int32)
    for r in range(_K):
        m = jnp.max(work, axis=1, keepdims=True)
        cand = jnp.where(work == m, lane, jnp.int32(1 << 30))
        j = jnp.min(cand, axis=1, keepdims=True)
        idxs = jnp.where(kcol == r, j, idxs)
        if r + 1 < _K:
            work = jnp.where(cand == j, _NEG, work)
    return idxs


def _layer(xt, P, Wt, Wd, d):
    S = xt.shape[0]
    ones = jnp.ones((S, 1), jnp.float32)
    xx = jnp.sum(xt * xt, axis=1, keepdims=True)
    A = jnp.concatenate([2.0 * xt, ones], axis=1)
    Bm = jnp.concatenate([xt, -xx], axis=1)
    pdp = _mm_t(A, Bm)
    idxs = _top16_idx(pdp)

    Y = jnp.dot(xt, Wt, preferred_element_type=jnp.float32)
    Z = jnp.dot(xt, Wd, preferred_element_type=jnp.float32)
    Yq, Yk, Yv = Y[:, :d], Y[:, d:2 * d], Y[:, 2 * d:]
    Zq, Zk, Zv = Z[:, :d], Z[:, d:2 * d], Z[:, 2 * d:]
    tab = jnp.concatenate([Yq * Yk, Yq, Yk, Yv], axis=1)
    lane = jax.lax.broadcasted_iota(jnp.int32, (S, S), 1)
    w1 = Zk + P
    inv = jnp.float32(1.0 / math.sqrt(d))
    logits_k = []
    yvg_k = []
    for r in range(_K):
        oh = (lane == idxs[:, r:r + 1]).astype(jnp.float32)
        g = jnp.dot(oh, tab, preferred_element_type=jnp.float32)
        logits_k.append((g[:, :d] + g[:, d:2 * d] * w1
                         + g[:, 2 * d:3 * d] * Zq) * inv)
        yvg_k.append(g[:, 3 * d:])
    mrow = logits_k[0]
    for r in range(1, _K):
        mrow = jnp.maximum(mrow, logits_k[r])
    es = [jnp.exp(lg - mrow) for lg in logits_k]
    ssum = es[0]
    for r in range(1, _K):
        ssum = ssum + es[r]
    acc = es[0] * yvg_k[0]
    for r in range(1, _K):
        acc = acc + es[r] * yvg_k[r]
    return acc / ssum + Zv + P


def _layers_body(pt_ref, wt1, wd1, wp1, wt2, wd2, wp2, wt3, wd3, wp3, out_ref):
    pt = pt_ref[0]
    P1 = jnp.dot(pt, wp1[...], preferred_element_type=jnp.float32)
    x1 = _layer(pt, P1, wt1[...], wd1[...], 64)
    P2 = jnp.dot(pt, wp2[...], preferred_element_type=jnp.float32)
    x2 = _layer(x1, P2, wt2[...], wd2[...], 64)
    P3 = jnp.dot(pt, wp3[...], preferred_element_type=jnp.float32)
    x3 = _layer(x2, P3, wt3[...], wd3[...], 128)
    feat = jnp.concatenate([x1, x2, x3], axis=1)
    mx = jnp.max(feat, axis=0)
    av = jnp.sum(feat, axis=0) * jnp.float32(1.0 / _S)
    out_ref[0, 0] = jnp.concatenate([mx, av])


def _prep_w(Wq, Wk, Wv, C, pad_to):
    top = jnp.concatenate([Wq[:C], Wk[:C], Wv[:C]], axis=1)
    dif = jnp.concatenate([Wq[C:] - Wq[:C], Wk[C:] - Wk[:C], Wv[C:] - Wv[:C]],
                          axis=1)
    if pad_to > C:
        z = jnp.zeros((pad_to - C, top.shape[1]), jnp.float32)
        top = jnp.concatenate([top, z], axis=0)
        dif = jnp.concatenate([dif, z], axis=0)
    return top, dif


def _pad_rows(W, pad_to):
    return jnp.concatenate(
        [W, jnp.zeros((pad_to - W.shape[0], W.shape[1]), jnp.float32)], axis=0)


def kernel(x, Wq1, Wk1, Wv1, Wp1, Wq2, Wk2, Wv2, Wp2, Wq3, Wk3, Wv3, Wp3):
    B, N, _ = x.shape
    planes = [x[:, :, c].reshape(B, 128, 128).transpose(1, 0, 2)
              for c in range(3)]
    o0, o1, o2 = pl.pallas_call(
        _fps_body,
        out_shape=[jax.ShapeDtypeStruct((1, _S, B), jnp.float32)] * 3,
        scratch_shapes=[pltpu.VMEM((128, B, 128), jnp.float32)],
        interpret=_INTERPRET,
    )(*planes)
    pt = jnp.stack([o0[0], o1[0], o2[0]], axis=-1).transpose(1, 0, 2)
    ptp = jnp.concatenate([pt, jnp.zeros((B, _S, 5), jnp.float32)], axis=-1)

    wt1, wd1 = _prep_w(Wq1, Wk1, Wv1, 3, 8)
    wt2, wd2 = _prep_w(Wq2, Wk2, Wv2, 64, 64)
    wt3, wd3 = _prep_w(Wq3, Wk3, Wv3, 64, 64)
    wp1 = _pad_rows(Wp1, 8)
    wp2 = _pad_rows(Wp2, 8)
    wp3 = _pad_rows(Wp3, 8)

    def wspec(w):
        return pl.BlockSpec(w.shape, lambda b: (0, 0))

    out = pl.pallas_call(
        _layers_body,
        grid=(B,),
        in_specs=[pl.BlockSpec((1, _S, 8), lambda b: (b, 0, 0)),
                  wspec(wt1), wspec(wd1), wspec(wp1),
                  wspec(wt2), wspec(wd2), wspec(wp2),
                  wspec(wt3), wspec(wd3), wspec(wp3)],
        out_specs=pl.BlockSpec((1, 1, _S), lambda b: (b, 0, 0)),
        out_shape=jax.ShapeDtypeStruct((B, 1, _S), jnp.float32),
        compiler_params=pltpu.CompilerParams(
            dimension_semantics=("arbitrary",)),
        interpret=_INTERPRET,
    )(ptp, wt1, wd1, wp1, wt2, wd2, wp2, wt3, wd3, wp3)
    return out.reshape(B, _S)


# speedup vs baseline: 1.3932x; 1.2216x over previous
import math

import jax
import jax.numpy as jnp
from jax.experimental import pallas as pl
from jax.experimental.pallas import tpu as pltpu

_S = 512
_K = 16
_NEG = float(-3e38)

_INTERPRET = False


def _mm_t(a, b):
    return jax.lax.dot_general(a, b, (((1,), (1,)), ((), ())),
                               preferred_element_type=jnp.float32)


def _fps_body(x0_ref, x1_ref, x2_ref, o0_ref, o1_ref, o2_ref, dists_ref):
    Bc = x0_ref.shape[1]
    X0 = x0_ref[...]
    X1 = x1_ref[...]
    X2 = x2_ref[...]
    ri = (jax.lax.broadcasted_iota(jnp.int32, (128, 1, 128), 0) * 128
          + jax.lax.broadcasted_iota(jnp.int32, (128, 1, 128), 2)
          ).astype(jnp.float32)
    dists_ref[...] = jnp.full((128, Bc, 128), 1e10, dtype=jnp.float32)

    UNROLL = 4

    def body(i, far):
        for r in range(UNROLL):
            msk = ri == far.reshape(1, Bc, 1)
            c0 = jnp.sum(jnp.sum(jnp.where(msk, X0, 0.0), axis=2), axis=0)
            c1 = jnp.sum(jnp.sum(jnp.where(msk, X1, 0.0), axis=2), axis=0)
            c2 = jnp.sum(jnp.sum(jnp.where(msk, X2, 0.0), axis=2), axis=0)
            step = i * UNROLL + r
            o0_ref[0, pl.ds(step, 1), :] = c0.reshape(1, Bc)
            o1_ref[0, pl.ds(step, 1), :] = c1.reshape(1, Bc)
            o2_ref[0, pl.ds(step, 1), :] = c2.reshape(1, Bc)
            d0 = X0 - c0.reshape(1, Bc, 1)
            d1 = X1 - c1.reshape(1, Bc, 1)
            d2 = X2 - c2.reshape(1, Bc, 1)
            d = d0 * d0 + d1 * d1 + d2 * d2
            nd = jnp.minimum(dists_ref[...], d)
            dists_ref[...] = nd
            mx = jnp.max(jnp.max(nd, axis=2), axis=0)
            cand = jnp.where(nd == mx.reshape(1, Bc, 1), ri,
                             jnp.float32(3e38))
            far = jnp.min(jnp.min(cand, axis=2), axis=0)
        return far

    jax.lax.fori_loop(0, _S // UNROLL, body, jnp.zeros((Bc,), jnp.float32))


def _top16_idx(pdp, lane):
    work = pdp
    js = []
    for r in range(_K):
        m = jnp.max(work, axis=1, keepdims=True)
        cand = jnp.where(work == m, lane, jnp.float32(3e38))
        j = jnp.min(cand, axis=1, keepdims=True)
        js.append(j)
        if r + 1 < _K:
            work = jnp.where(cand == j, _NEG, work)
    return js


def _layer(xt, P, Wt, Wd, d):
    S = xt.shape[0]
    ones = jnp.ones((S, 1), jnp.float32)
    xx = jnp.sum(xt * xt, axis=1, keepdims=True)
    A = jnp.concatenate([2.0 * xt, ones], axis=1)
    Bm = jnp.concatenate([xt, -xx], axis=1)
    pdp = _mm_t(A, Bm)
    lane = jax.lax.broadcasted_iota(jnp.int32, (S, S), 1).astype(jnp.float32)
    js = _top16_idx(pdp, lane)

    Y = jnp.dot(xt, Wt, preferred_element_type=jnp.float32)
    Z = jnp.dot(xt, Wd, preferred_element_type=jnp.float32)
    Yq, Yk, Yv = Y[:, :d], Y[:, d:2 * d], Y[:, 2 * d:]
    Zq, Zk, Zv = Z[:, :d], Z[:, d:2 * d], Z[:, 2 * d:]
    tab = jnp.concatenate([Yq * Yk, Yq, Yk, Yv], axis=1)
    w1 = Zk + P
    inv = jnp.float32(1.0 / math.sqrt(d))
    logits_k = []
    yvg_k = []
    for r in range(_K):
        oh = jnp.where(lane == js[r], 1.0, 0.0)
        g = jnp.dot(oh, tab, preferred_element_type=jnp.float32)
        logits_k.append((g[:, :d] + g[:, d:2 * d] * w1
                         + g[:, 2 * d:3 * d] * Zq) * inv)
        yvg_k.append(g[:, 3 * d:])
    mrow = logits_k[0]
    for r in range(1, _K):
        mrow = jnp.maximum(mrow, logits_k[r])
    es = [jnp.exp(lg - mrow) for lg in logits_k]
    ssum = es[0]
    for r in range(1, _K):
        ssum = ssum + es[r]
    acc = es[0] * yvg_k[0]
    for r in range(1, _K):
        acc = acc + es[r] * yvg_k[r]
    return acc / ssum + Zv + P


def _layers_body(pt_ref, wt1, wd1, wp1, wt2, wd2, wp2, wt3, wd3, wp3, out_ref):
    pt = pt_ref[0]
    P1 = jnp.dot(pt, wp1[...], preferred_element_type=jnp.float32)
    x1 = _layer(pt, P1, wt1[...], wd1[...], 64)
    P2 = jnp.dot(pt, wp2[...], preferred_element_type=jnp.float32)
    x2 = _layer(x1, P2, wt2[...], wd2[...], 64)
    P3 = jnp.dot(pt, wp3[...], preferred_element_type=jnp.float32)
    x3 = _layer(x2, P3, wt3[...], wd3[...], 128)
    feat = jnp.concatenate([x1, x2, x3], axis=1)
    mx = jnp.max(feat, axis=0)
    av = jnp.sum(feat, axis=0) * jnp.float32(1.0 / _S)
    out_ref[0, 0] = jnp.concatenate([mx, av])


def _prep_w(Wq, Wk, Wv, C, pad_to):
    top = jnp.concatenate([Wq[:C], Wk[:C], Wv[:C]], axis=1)
    dif = jnp.concatenate([Wq[C:] - Wq[:C], Wk[C:] - Wk[:C], Wv[C:] - Wv[:C]],
                          axis=1)
    if pad_to > C:
        z = jnp.zeros((pad_to - C, top.shape[1]), jnp.float32)
        top = jnp.concatenate([top, z], axis=0)
        dif = jnp.concatenate([dif, z], axis=0)
    return top, dif


def _pad_rows(W, pad_to):
    return jnp.concatenate(
        [W, jnp.zeros((pad_to - W.shape[0], W.shape[1]), jnp.float32)], axis=0)


def kernel(x, Wq1, Wk1, Wv1, Wp1, Wq2, Wk2, Wv2, Wp2, Wq3, Wk3, Wv3, Wp3):
    B, N, _ = x.shape
    planes = [x[:, :, c].reshape(B, 128, 128).transpose(1, 0, 2)
              for c in range(3)]
    o0, o1, o2 = pl.pallas_call(
        _fps_body,
        out_shape=[jax.ShapeDtypeStruct((1, _S, B), jnp.float32)] * 3,
        scratch_shapes=[pltpu.VMEM((128, B, 128), jnp.float32)],
        interpret=_INTERPRET,
    )(*planes)
    pt = jnp.stack([o0[0], o1[0], o2[0]], axis=-1).transpose(1, 0, 2)
    ptp = jnp.concatenate([pt, jnp.zeros((B, _S, 5), jnp.float32)], axis=-1)

    wt1, wd1 = _prep_w(Wq1, Wk1, Wv1, 3, 8)
    wt2, wd2 = _prep_w(Wq2, Wk2, Wv2, 64, 64)
    wt3, wd3 = _prep_w(Wq3, Wk3, Wv3, 64, 64)
    wp1 = _pad_rows(Wp1, 8)
    wp2 = _pad_rows(Wp2, 8)
    wp3 = _pad_rows(Wp3, 8)

    def wspec(w):
        return pl.BlockSpec(w.shape, lambda b: (0, 0))

    out = pl.pallas_call(
        _layers_body,
        grid=(B,),
        in_specs=[pl.BlockSpec((1, _S, 8), lambda b: (b, 0, 0)),
                  wspec(wt1), wspec(wd1), wspec(wp1),
                  wspec(wt2), wspec(wd2), wspec(wp2),
                  wspec(wt3), wspec(wd3), wspec(wp3)],
        out_specs=pl.BlockSpec((1, 1, _S), lambda b: (b, 0, 0)),
        out_shape=jax.ShapeDtypeStruct((B, 1, _S), jnp.float32),
        compiler_params=pltpu.CompilerParams(
            dimension_semantics=("arbitrary",)),
        interpret=_INTERPRET,
    )(ptp, wt1, wd1, wp1, wt2, wd2, wp2, wt3, wd3, wp3)
    return out.reshape(B, _S)


# speedup vs baseline: 1.4551x; 1.0444x over previous
import math

import jax
import jax.numpy as jnp
from jax.experimental import pallas as pl
from jax.experimental.pallas import tpu as pltpu

_S = 512
_K = 16
_NEG = float(-3e38)

_INTERPRET = False


def _mm_t(a, b):
    return jax.lax.dot_general(a, b, (((1,), (1,)), ((), ())),
                               preferred_element_type=jnp.float32)


def _fps_body(x0_ref, x1_ref, x2_ref, o0_ref, o1_ref, o2_ref, dists_ref):
    Bc = x0_ref.shape[1]
    X0 = x0_ref[...]
    X1 = x1_ref[...]
    X2 = x2_ref[...]
    ri = (jax.lax.broadcasted_iota(jnp.int32, (128, 1, 128), 0) * 128
          + jax.lax.broadcasted_iota(jnp.int32, (128, 1, 128), 2)
          ).astype(jnp.float32)
    dists_ref[...] = jnp.full((128, Bc, 128), 1e10, dtype=jnp.float32)

    UNROLL = 4

    def body(i, far):
        for r in range(UNROLL):
            msk = ri == far.reshape(1, Bc, 1)
            c0 = jnp.sum(jnp.sum(jnp.where(msk, X0, 0.0), axis=0), axis=1)
            c1 = jnp.sum(jnp.sum(jnp.where(msk, X1, 0.0), axis=0), axis=1)
            c2 = jnp.sum(jnp.sum(jnp.where(msk, X2, 0.0), axis=0), axis=1)
            step = i * UNROLL + r
            o0_ref[0, pl.ds(step, 1), :] = c0.reshape(1, Bc)
            o1_ref[0, pl.ds(step, 1), :] = c1.reshape(1, Bc)
            o2_ref[0, pl.ds(step, 1), :] = c2.reshape(1, Bc)
            d0 = X0 - c0.reshape(1, Bc, 1)
            d1 = X1 - c1.reshape(1, Bc, 1)
            d2 = X2 - c2.reshape(1, Bc, 1)
            d = d0 * d0 + d1 * d1 + d2 * d2
            nd = jnp.minimum(dists_ref[...], d)
            dists_ref[...] = nd
            mx = jnp.max(jnp.max(nd, axis=0), axis=1)
            cand = jnp.where(nd == mx.reshape(1, Bc, 1), ri,
                             jnp.float32(3e38))
            far = jnp.min(jnp.min(cand, axis=0), axis=1)
        return far

    jax.lax.fori_loop(0, _S // UNROLL, body, jnp.zeros((Bc,), jnp.float32))


def _top16_idx(pdp, lane):
    work = pdp
    js = []
    for r in range(_K):
        m = jnp.max(work, axis=1, keepdims=True)
        cand = jnp.where(work == m, lane, jnp.float32(3e38))
        j = jnp.min(cand, axis=1, keepdims=True)
        js.append(j)
        if r + 1 < _K:
            work = jnp.where(cand == j, _NEG, work)
    return js


def _layer(xt, P, Wt, Wd, d, lane):
    S = xt.shape[0]
    ones = jnp.ones((S, 1), jnp.float32)
    xx = jnp.sum(xt * xt, axis=1, keepdims=True)
    A = jnp.concatenate([2.0 * xt, ones], axis=1)
    Bm = jnp.concatenate([xt, -xx], axis=1)
    pdp = _mm_t(A, Bm)
    js = _top16_idx(pdp, lane)

    Y = jnp.dot(xt, Wt, preferred_element_type=jnp.float32)
    Z = jnp.dot(xt, Wd, preferred_element_type=jnp.float32)
    Yq, Yk, Yv = Y[:, :d], Y[:, d:2 * d], Y[:, 2 * d:]
    Zq, Zk, Zv = Z[:, :d], Z[:, d:2 * d], Z[:, 2 * d:]
    tab = jnp.concatenate([Yq * Yk, Yq, Yk, Yv], axis=1)
    w1 = Zk + P
    inv = jnp.float32(1.0 / math.sqrt(d))
    logits_k = []
    yvg_k = []
    for r in range(_K):
        oh = jnp.where(lane == js[r], 1.0, 0.0)
        g = jnp.dot(oh, tab, preferred_element_type=jnp.float32)
        logits_k.append((g[:, :d] + g[:, d:2 * d] * w1
                         + g[:, 2 * d:3 * d] * Zq) * inv)
        yvg_k.append(g[:, 3 * d:])
    mrow = logits_k[0]
    for r in range(1, _K):
        mrow = jnp.maximum(mrow, logits_k[r])
    es = [jnp.exp(lg - mrow) for lg in logits_k]
    ssum = es[0]
    for r in range(1, _K):
        ssum = ssum + es[r]
    acc = es[0] * yvg_k[0]
    for r in range(1, _K):
        acc = acc + es[r] * yvg_k[r]
    return acc / ssum + Zv + P


def _layers_body(pt_ref, wt1, wd1, wp1, wt2, wd2, wp2, wt3, wd3, wp3, out_ref):
    pt = pt_ref[0]
    lane = jax.lax.broadcasted_iota(jnp.int32, (_S, _S), 1).astype(jnp.float32)
    P1 = jnp.dot(pt, wp1[...], preferred_element_type=jnp.float32)
    x1 = _layer(pt, P1, wt1[...], wd1[...], 64, lane)
    P2 = jnp.dot(pt, wp2[...], preferred_element_type=jnp.float32)
    x2 = _layer(x1, P2, wt2[...], wd2[...], 64, lane)
    P3 = jnp.dot(pt, wp3[...], preferred_element_type=jnp.float32)
    x3 = _layer(x2, P3, wt3[...], wd3[...], 128, lane)
    feat = jnp.concatenate([x1, x2, x3], axis=1)
    mx = jnp.max(feat, axis=0)
    av = jnp.sum(feat, axis=0) * jnp.float32(1.0 / _S)
    out_ref[0, 0] = jnp.concatenate([mx, av])


def _prep_w(Wq, Wk, Wv, C, pad_to):
    top = jnp.concatenate([Wq[:C], Wk[:C], Wv[:C]], axis=1)
    dif = jnp.concatenate([Wq[C:] - Wq[:C], Wk[C:] - Wk[:C], Wv[C:] - Wv[:C]],
                          axis=1)
    if pad_to > C:
        z = jnp.zeros((pad_to - C, top.shape[1]), jnp.float32)
        top = jnp.concatenate([top, z], axis=0)
        dif = jnp.concatenate([dif, z], axis=0)
    return top, dif


def _pad_rows(W, pad_to):
    return jnp.concatenate(
        [W, jnp.zeros((pad_to - W.shape[0], W.shape[1]), jnp.float32)], axis=0)


def kernel(x, Wq1, Wk1, Wv1, Wp1, Wq2, Wk2, Wv2, Wp2, Wq3, Wk3, Wv3, Wp3):
    B, N, _ = x.shape
    planes = [x[:, :, c].reshape(B, 128, 128).transpose(1, 0, 2)
              for c in range(3)]
    o0, o1, o2 = pl.pallas_call(
        _fps_body,
        out_shape=[jax.ShapeDtypeStruct((1, _S, B), jnp.float32)] * 3,
        scratch_shapes=[pltpu.VMEM((128, B, 128), jnp.float32)],
        interpret=_INTERPRET,
    )(*planes)
    pt = jnp.stack([o0[0], o1[0], o2[0]], axis=-1).transpose(1, 0, 2)
    ptp = jnp.concatenate([pt, jnp.zeros((B, _S, 5), jnp.float32)], axis=-1)

    wt1, wd1 = _prep_w(Wq1, Wk1, Wv1, 3, 8)
    wt2, wd2 = _prep_w(Wq2, Wk2, Wv2, 64, 64)
    wt3, wd3 = _prep_w(Wq3, Wk3, Wv3, 64, 64)
    wp1 = _pad_rows(Wp1, 8)
    wp2 = _pad_rows(Wp2, 8)
    wp3 = _pad_rows(Wp3, 8)

    def wspec(w):
        return pl.BlockSpec(w.shape, lambda b: (0, 0))

    out = pl.pallas_call(
        _layers_body,
        grid=(B,),
        in_specs=[pl.BlockSpec((1, _S, 8), lambda b: (b, 0, 0)),
                  wspec(wt1), wspec(wd1), wspec(wp1),
                  wspec(wt2), wspec(wd2), wspec(wp2),
                  wspec(wt3), wspec(wd3), wspec(wp3)],
        out_specs=pl.BlockSpec((1, 1, _S), lambda b: (b, 0, 0)),
        out_shape=jax.ShapeDtypeStruct((B, 1, _S), jnp.float32),
        compiler_params=pltpu.CompilerParams(
            dimension_semantics=("arbitrary",)),
        interpret=_INTERPRET,
    )(ptp, wt1, wd1, wp1, wt2, wd2, wp2, wt3, wd3, wp3)
    return out.reshape(B, _S)
